# initial kernel scaffold (unmeasured)
import jax
import jax.numpy as jnp
from jax import lax
from jax.experimental import pallas as pl
from jax.experimental.pallas import tpu as pltpu

N_DEV = 4


def kernel(x, w_mat, scale_x, scale_w):
    m_total, k_shard = x.shape
    k_total, n = w_mat.shape
    m_out = m_total // N_DEV
    half_n = n // 2
    n_chunk = 1024

    s = (scale_x.astype(jnp.float32) * scale_w.astype(jnp.float32)).reshape(1, 1)

    def body(x_ref, w_hbm, s_ref, out_ref, comm_ref, wbuf, send_sems, recv_sems, w_sems):
        my = lax.axis_index("i")

        barrier = pltpu.get_barrier_semaphore()
        for r in (1, 2, 3):
            pl.semaphore_signal(
                barrier, inc=1,
                device_id=(lax.rem(my + r, N_DEV),),
                device_id_type=pl.DeviceIdType.MESH,
            )
        pl.semaphore_wait(barrier, N_DEV - 1)

        sends = []
        for r in (1, 2, 3):
            d = lax.rem(my + r, N_DEV)
            rdma = pltpu.make_async_remote_copy(
                src_ref=x_ref.at[pl.ds(d * m_out, m_out), :],
                dst_ref=comm_ref.at[r],
                send_sem=send_sems.at[r - 1],
                recv_sem=recv_sems.at[r],
                device_id=(d,),
                device_id_type=pl.DeviceIdType.MESH,
            )
            rdma.start()
            sends.append(rdma)

        order = [
            (0, my),
            (1, lax.rem(my + 3, N_DEV)),
            (3, lax.rem(my + 1, N_DEV)),
            (2, lax.rem(my + 2, N_DEV)),
        ]

        def w_dma(j, half, slot):
            return pltpu.make_async_copy(
                w_hbm.at[pl.ds(j * m_out, m_out), pl.ds(half * half_n, half_n)],
                wbuf.at[slot],
                w_sems.at[slot],
            )

        steps = [(ti, half) for ti in range(N_DEV) for half in range(2)]
        w_dma(order[0][1], 0, 0).start()

        for t, (ti, half) in enumerate(steps):
            r, j = order[ti]
            slot = t % 2
            w_dma(j, half, slot).wait()
            if t + 1 < len(steps):
                ti2, half2 = steps[t + 1]
                w_dma(order[ti2][1], half2, (t + 1) % 2).start()

            if ti > 0 and half == 0:
                pltpu.make_async_remote_copy(
                    src_ref=comm_ref.at[r],
                    dst_ref=comm_ref.at[r],
                    send_sem=send_sems.at[0],
                    recv_sem=recv_sems.at[r],
                    device_id=(my,),
                    device_id_type=pl.DeviceIdType.MESH,
                ).wait_recv()

            if ti == 0:
                xblk = x_ref[pl.ds(my * m_out, m_out), :]
            else:
                xblk = comm_ref[r]

            for c in range(half_n // n_chunk):
                col = half * half_n + c * n_chunk
                dot = jnp.dot(
                    xblk,
                    wbuf[slot][:, c * n_chunk:(c + 1) * n_chunk],
                    preferred_element_type=jnp.float32,
                )
                sl = pl.ds(col, n_chunk)
                if ti == 0:
                    out_ref[:, sl] = dot
                elif ti < N_DEV - 1:
                    out_ref[:, sl] = out_ref[:, sl] + dot
                else:
                    out_ref[:, sl] = (out_ref[:, sl] + dot) * s_ref[0, 0]

        for rdma in sends:
            rdma.wait_send()

    return pl.pallas_call(
        body,
        out_shape=jax.ShapeDtypeStruct((m_out, n), jnp.float32),
        in_specs=[
            pl.BlockSpec(memory_space=pltpu.VMEM),
            pl.BlockSpec(memory_space=pltpu.ANY),
            pl.BlockSpec(memory_space=pltpu.SMEM),
        ],
        out_specs=pl.BlockSpec(memory_space=pltpu.VMEM),
        scratch_shapes=[
            pltpu.VMEM((N_DEV, m_out, k_shard), x.dtype),
            pltpu.VMEM((2, m_out, half_n), w_mat.dtype),
            pltpu.SemaphoreType.DMA((3,)),
            pltpu.SemaphoreType.DMA((N_DEV,)),
            pltpu.SemaphoreType.DMA((2,)),
        ],
        compiler_params=pltpu.CompilerParams(collective_id=0),
    )(x, w_mat, s)


# baseline (device time: 116030 ns/iter reference)
import jax
import jax.numpy as jnp
from jax import lax
from jax.experimental import pallas as pl
from jax.experimental.pallas import tpu as pltpu

N_DEV = 4
FP8 = jnp.float8_e5m2


def kernel(x, w_mat, scale_x, scale_w):
    m_total, k_shard = x.shape
    k_total, n = w_mat.shape
    m_out = m_total // N_DEV
    n_pieces = n // k_shard

    s = (scale_x.astype(jnp.float32) * scale_w.astype(jnp.float32)).reshape(1, 1)

    def body(x_hbm, w_hbm, s_ref, out_ref,
             xf32, xq, comm_ref, wbuf, x_sems, w_sems, send_sems, recv_sems):
        my = lax.axis_index("i")

        barrier = pltpu.get_barrier_semaphore()
        for r in (1, 2, 3):
            pl.semaphore_signal(
                barrier, inc=1,
                device_id=(lax.rem(my + r, N_DEV),),
                device_id_type=pl.DeviceIdType.MESH,
            )
        pl.semaphore_wait(barrier, N_DEV - 1)

        def x_dma(d, slot):
            return pltpu.make_async_copy(
                x_hbm.at[pl.ds(d * m_out, m_out), :], xf32.at[slot], x_sems.at[slot],
            )

        def w_dma(j, p, slot):
            return pltpu.make_async_copy(
                w_hbm.at[pl.ds(j * m_out, m_out), pl.ds(p * k_shard, k_shard)],
                wbuf.at[slot],
                w_sems.at[slot],
            )

        d_order = [(lax.rem(my + 1, N_DEV), 1),
                   (lax.rem(my + 3, N_DEV), 3),
                   (lax.rem(my + 2, N_DEV), 2),
                   (my, 0)]
        x_dma(d_order[0][0], 0).start()

        order = [
            (0, my),
            (1, lax.rem(my + 3, N_DEV)),
            (3, lax.rem(my + 1, N_DEV)),
            (2, lax.rem(my + 2, N_DEV)),
        ]
        w_dma(order[0][1], 0, 0).start()

        sends = []
        for i, (d, r) in enumerate(d_order):
            x_dma(d, i % 2).wait()
            if i + 1 < N_DEV:
                x_dma(d_order[i + 1][0], (i + 1) % 2).start()
            xq[r] = xf32[i % 2].astype(FP8)
            if r != 0:
                rdma = pltpu.make_async_remote_copy(
                    src_ref=xq.at[r],
                    dst_ref=comm_ref.at[r],
                    send_sem=send_sems.at[r - 1],
                    recv_sem=recv_sems.at[r],
                    device_id=(d,),
                    device_id_type=pl.DeviceIdType.MESH,
                )
                rdma.start()
                sends.append(rdma)

        steps = [(ti, p) for ti in range(N_DEV) for p in range(n_pieces)]
        for t, (ti, p) in enumerate(steps):
            r, j = order[ti]
            slot = t % 2
            w_dma(j, p, slot).wait()
            if t + 1 < len(steps):
                ti2, p2 = steps[t + 1]
                w_dma(order[ti2][1], p2, (t + 1) % 2).start()

            if ti > 0 and p == 0:
                pltpu.make_async_remote_copy(
                    src_ref=comm_ref.at[r],
                    dst_ref=comm_ref.at[r],
                    send_sem=send_sems.at[0],
                    recv_sem=recv_sems.at[r],
                    device_id=(my,),
                    device_id_type=pl.DeviceIdType.MESH,
                ).wait_recv()

            xblk = xq[0] if ti == 0 else comm_ref[r]
            dot = jnp.dot(xblk, wbuf[slot].astype(FP8),
                          preferred_element_type=jnp.float32)
            sl = pl.ds(p * k_shard, k_shard)
            if ti == 0:
                out_ref[:, sl] = dot
            elif ti < N_DEV - 1:
                out_ref[:, sl] = out_ref[:, sl] + dot
            else:
                out_ref[:, sl] = (out_ref[:, sl] + dot) * s_ref[0, 0]

        for rdma in sends:
            rdma.wait_send()

    return pl.pallas_call(
        body,
        out_shape=jax.ShapeDtypeStruct((m_out, n), jnp.float32),
        in_specs=[
            pl.BlockSpec(memory_space=pl.ANY),
            pl.BlockSpec(memory_space=pl.ANY),
            pl.BlockSpec(memory_space=pltpu.SMEM),
        ],
        out_specs=pl.BlockSpec(memory_space=pltpu.VMEM),
        scratch_shapes=[
            pltpu.VMEM((2, m_out, k_shard), jnp.float32),
            pltpu.VMEM((N_DEV, m_out, k_shard), FP8),
            pltpu.VMEM((N_DEV, m_out, k_shard), FP8),
            pltpu.VMEM((2, m_out, k_shard), jnp.float32),
            pltpu.SemaphoreType.DMA((2,)),
            pltpu.SemaphoreType.DMA((2,)),
            pltpu.SemaphoreType.DMA((3,)),
            pltpu.SemaphoreType.DMA((N_DEV,)),
        ],
        compiler_params=pltpu.CompilerParams(
            collective_id=0, vmem_limit_bytes=63 * 1024 * 1024,
        ),
    )(x, w_mat, s)


# device time: 113826 ns/iter; 1.0194x vs baseline; 1.0194x over previous
import jax
import jax.numpy as jnp
from jax import lax
from jax.experimental import pallas as pl
from jax.experimental.pallas import tpu as pltpu

N_DEV = 4
FP8 = jnp.float8_e5m2
K_STRIP = 256
N_CHUNK = 1024


def kernel(x, w_mat, scale_x, scale_w):
    m_total, k_shard = x.shape
    k_total, n = w_mat.shape
    m_out = m_total // N_DEV
    n_strips = m_out // K_STRIP
    n_chunks = n // N_CHUNK

    s = (scale_x.astype(jnp.float32) * scale_w.astype(jnp.float32)).reshape(1, 1)
    xq = x.astype(FP8)

    def body(x_ref, w_hbm, s_ref, out_ref, comm_ref, wbuf, w_sems, send_sems, recv_sems):
        my = lax.axis_index("i")

        barrier = pltpu.get_barrier_semaphore()
        for r in (1, 2, 3):
            pl.semaphore_signal(
                barrier, inc=1,
                device_id=(lax.rem(my + r, N_DEV),),
                device_id_type=pl.DeviceIdType.MESH,
            )
        pl.semaphore_wait(barrier, N_DEV - 1)

        sends = []
        for r in (1, 3, 2):
            d = lax.rem(my + r, N_DEV)
            rdma = pltpu.make_async_remote_copy(
                src_ref=x_ref.at[pl.ds(d * m_out, m_out), :],
                dst_ref=comm_ref.at[r],
                send_sem=send_sems.at[r - 1],
                recv_sem=recv_sems.at[r],
                device_id=(d,),
                device_id_type=pl.DeviceIdType.MESH,
            )
            rdma.start()
            sends.append(rdma)

        order = [
            (0, my),
            (1, lax.rem(my + 3, N_DEV)),
            (3, lax.rem(my + 1, N_DEV)),
            (2, lax.rem(my + 2, N_DEV)),
        ]

        def w_dma(j, strip, slot):
            return pltpu.make_async_copy(
                w_hbm.at[pl.ds(j * m_out + strip * K_STRIP, K_STRIP), :],
                wbuf.at[slot],
                w_sems.at[slot],
            )

        steps = [(ti, st) for ti in range(N_DEV) for st in range(n_strips)]
        w_dma(order[0][1], 0, 0).start()

        for t, (ti, st) in enumerate(steps):
            r, j = order[ti]
            slot = t % 2
            w_dma(j, st, slot).wait()
            if t + 1 < len(steps):
                ti2, st2 = steps[t + 1]
                w_dma(order[ti2][1], st2, (t + 1) % 2).start()

            if ti > 0 and st == 0:
                pltpu.make_async_remote_copy(
                    src_ref=comm_ref.at[r],
                    dst_ref=comm_ref.at[r],
                    send_sem=send_sems.at[0],
                    recv_sem=recv_sems.at[r],
                    device_id=(my,),
                    device_id_type=pl.DeviceIdType.MESH,
                ).wait_recv()

            ks = pl.ds(st * K_STRIP, K_STRIP)
            xs = x_ref[pl.ds(my * m_out, m_out), ks] if ti == 0 else comm_ref[r, :, ks]
            for c in range(n_chunks):
                dot = jnp.dot(
                    xs,
                    wbuf[slot][:, c * N_CHUNK:(c + 1) * N_CHUNK].astype(FP8),
                    preferred_element_type=jnp.float32,
                )
                sl = pl.ds(c * N_CHUNK, N_CHUNK)
                if t == 0:
                    out_ref[:, sl] = dot
                elif t < len(steps) - 1:
                    out_ref[:, sl] = out_ref[:, sl] + dot
                else:
                    out_ref[:, sl] = (out_ref[:, sl] + dot) * s_ref[0, 0]

        for rdma in sends:
            rdma.wait_send()

    return pl.pallas_call(
        body,
        out_shape=jax.ShapeDtypeStruct((m_out, n), jnp.float32),
        in_specs=[
            pl.BlockSpec(memory_space=pltpu.VMEM),
            pl.BlockSpec(memory_space=pl.ANY),
            pl.BlockSpec(memory_space=pltpu.SMEM),
        ],
        out_specs=pl.BlockSpec(memory_space=pltpu.VMEM),
        scratch_shapes=[
            pltpu.VMEM((N_DEV, m_out, k_shard), FP8),
            pltpu.VMEM((2, K_STRIP, n), jnp.float32),
            pltpu.SemaphoreType.DMA((2,)),
            pltpu.SemaphoreType.DMA((3,)),
            pltpu.SemaphoreType.DMA((N_DEV,)),
        ],
        compiler_params=pltpu.CompilerParams(
            collective_id=0, vmem_limit_bytes=63 * 1024 * 1024,
        ),
    )(xq, w_mat, s)
